# baseline (speedup 1.0000x reference)
import jax
import jax.numpy as jnp
from jax import lax
from jax.experimental import pallas as pl
from jax.experimental.pallas import tpu as pltpu
from jax.experimental.pallas import tpu_sc as plsc

N = 10000
E = 160000
D = 256
NC = 2
NS = 16
L = 16
SLAB = 1280
NPASS = 4
NPAD = NC * NPASS * SLAB
EPT = E // NS
STRIPE = SLAB // NS
CH = 64
CAP = EPT + CH


def _leaky(x):
    return jnp.where(x > 0, x, 0.3 * x)


def _mm1_body(x_ref, a_ref, o_ref):
    o_ref[...] = _leaky(
        jnp.dot(x_ref[...], a_ref[...], preferred_element_type=jnp.float32)
    )


def _stage1(h_src, Q):
    BR = 512
    return pl.pallas_call(
        _mm1_body,
        grid=(pl.cdiv(N, BR),),
        in_specs=[
            pl.BlockSpec((BR, D), lambda i: (i, 0)),
            pl.BlockSpec((D, D), lambda i: (0, 0)),
        ],
        out_specs=pl.BlockSpec((BR, D), lambda i: (i, 0)),
        out_shape=jax.ShapeDtypeStruct((N, D), jnp.float32),
    )(h_src, Q)


def _mm2_body(n_ref, ws_ref, h_ref, w1_ref, w2_ref, o_ref):
    r = n_ref[...] / jnp.maximum(ws_ref[...], 1.0)
    y = jnp.dot(r, w1_ref[...], preferred_element_type=jnp.float32)
    y = y + jnp.dot(h_ref[...], w2_ref[...], preferred_element_type=jnp.float32)
    o_ref[...] = _leaky(y)


def _stage3(n, ws256, h_dst, W1, W2):
    BR = 512
    return pl.pallas_call(
        _mm2_body,
        grid=(pl.cdiv(N, BR),),
        in_specs=[
            pl.BlockSpec((BR, D), lambda i: (i, 0)),
            pl.BlockSpec((BR, D), lambda i: (i, 0)),
            pl.BlockSpec((BR, D), lambda i: (i, 0)),
            pl.BlockSpec((D, D), lambda i: (0, 0)),
            pl.BlockSpec((D, D), lambda i: (0, 0)),
        ],
        out_specs=pl.BlockSpec((BR, D), lambda i: (i, 0)),
        out_shape=jax.ShapeDtypeStruct((N, D), jnp.float32),
    )(n, ws256, h_dst, W1, W2)


def kernel(h_src, h_dst, edge_index, weights, Q, W):
    src = edge_index[0]
    dst = edge_index[1]
    w = weights.astype(jnp.float32)
    n_src = _stage1(h_src, Q)
    m = n_src[src] * w[:, None]
    n = jax.ops.segment_sum(m, dst, num_segments=N)
    ws = jax.ops.segment_sum(w, dst, num_segments=N)
    ws256 = jnp.tile(ws[:, None], (1, D))
    z = _stage3(n, ws256, h_dst, W[:D], W[D:])
    return z
